# baseline (speedup 1.0000x reference)
import jax
import jax.numpy as jnp
from jax.experimental import pallas as pl
from jax.experimental.pallas import tpu as pltpu

_N_STATES = 4
_N_ACTIONS = 2
_HID = 64
_BLOCK_L = 65536


def _mlp_t_kernel(xT_ref, w1_ref, b1_ref, w2_ref, b2_ref, oT_ref):
    h = jnp.dot(w1_ref[...], xT_ref[...],
                preferred_element_type=jnp.float32).astype(jnp.bfloat16)
    h = jnp.maximum(h + b1_ref[...], jnp.bfloat16(0.0))
    oT_ref[...] = (
        jnp.dot(w2_ref[...], h, preferred_element_type=jnp.float32)
        + b2_ref[...]
    )


def kernel(x, w1p, b1p, w2p, b2p):
    B = x.shape[0]
    xT = jnp.transpose(x).astype(jnp.bfloat16)
    w1s = w1p[:_HID, :].astype(jnp.bfloat16)
    b1s = b1p[:_HID, :].astype(jnp.bfloat16)
    w2s = w2p[:_N_ACTIONS, :_HID].astype(jnp.bfloat16)
    b2s = b2p[:_N_ACTIONS, :]

    block_l = min(_BLOCK_L, B)
    num_blocks = B // block_l

    oT = pl.pallas_call(
        _mlp_t_kernel,
        out_shape=jax.ShapeDtypeStruct((_N_ACTIONS, B), jnp.float32),
        grid=(num_blocks,),
        in_specs=[
            pl.BlockSpec((_N_STATES, block_l), lambda i: (0, i)),
            pl.BlockSpec((_HID, _N_STATES), lambda i: (0, 0)),
            pl.BlockSpec((_HID, 1), lambda i: (0, 0)),
            pl.BlockSpec((_N_ACTIONS, _HID), lambda i: (0, 0)),
            pl.BlockSpec((_N_ACTIONS, 1), lambda i: (0, 0)),
        ],
        out_specs=pl.BlockSpec((_N_ACTIONS, block_l), lambda i: (0, i)),
        compiler_params=pltpu.CompilerParams(
            dimension_semantics=("parallel",)),
    )(xT, w1s, b1s, w2s, b2s)
    return jnp.transpose(oT)


# speedup vs baseline: 1.1788x; 1.1788x over previous
import jax
import jax.numpy as jnp
from jax.experimental import pallas as pl
from jax.experimental.pallas import tpu as pltpu

_N_STATES = 4
_N_ACTIONS = 2
_HID = 64
_BLOCK_L = 131072


def _mlp_t_kernel(xT_ref, w1_ref, b1_ref, w2_ref, b2_ref, oT_ref):
    h = jnp.maximum(
        jnp.dot(w1_ref[...], xT_ref[...], preferred_element_type=jnp.float32)
        + b1_ref[...],
        0.0,
    )
    oT_ref[...] = (
        jnp.dot(w2_ref[...], h, preferred_element_type=jnp.float32)
        + b2_ref[...]
    )


def kernel(x, w1p, b1p, w2p, b2p):
    B = x.shape[0]
    xT = jnp.transpose(x)
    w1s = w1p[:_HID, :]
    b1s = b1p[:_HID, :]
    w2s = w2p[:_N_ACTIONS, :_HID]
    b2s = b2p[:_N_ACTIONS, :]

    block_l = min(_BLOCK_L, B)
    num_blocks = B // block_l

    oT = pl.pallas_call(
        _mlp_t_kernel,
        out_shape=jax.ShapeDtypeStruct((_N_ACTIONS, B), jnp.float32),
        grid=(num_blocks,),
        in_specs=[
            pl.BlockSpec((_N_STATES, block_l), lambda i: (0, i)),
            pl.BlockSpec((_HID, _N_STATES), lambda i: (0, 0)),
            pl.BlockSpec((_HID, 1), lambda i: (0, 0)),
            pl.BlockSpec((_N_ACTIONS, _HID), lambda i: (0, 0)),
            pl.BlockSpec((_N_ACTIONS, 1), lambda i: (0, 0)),
        ],
        out_specs=pl.BlockSpec((_N_ACTIONS, block_l), lambda i: (0, i)),
        compiler_params=pltpu.CompilerParams(
            dimension_semantics=("parallel",)),
    )(xT, w1s, b1s, w2s, b2s)
    return jnp.transpose(oT)
